# baseline (speedup 1.0000x reference)
import jax
import jax.numpy as jnp
from jax import lax
from jax.experimental import pallas as pl
from jax.experimental.pallas import tpu as pltpu
from jax.experimental.pallas import tpu_sc as plsc

ENT_MAX_LEN = 20
REL_MAX_LEN = 20
DIM = 128
BATCH = 4096

NUM_CORES = 2
NUM_SUBCORES = 16
NW = NUM_CORES * NUM_SUBCORES
BW = BATCH // NW
NBUF = 6
NRELS = 1000
RPT = 64
L = 16


def _clamp_ids(ref_1d, n, hi):
  for c in range(n // L):
    v = ref_1d[pl.ds(c * L, L)]
    ref_1d[pl.ds(c * L, L)] = jnp.minimum(jnp.maximum(v, 0), hi)


def _clamp_tok(tok, n_tok, hi):
  for off in (0, n_tok - L):
    v = tok[pl.ds(off, L)]
    tok[pl.ds(off, L)] = jnp.minimum(jnp.maximum(v, 0), hi)


def _accum_elem(rows, n_tok, acc_v, i, col0):
  for c in range(DIM // L):
    s = rows[0, pl.ds(c * L, L)]
    for t in range(1, n_tok):
      s = s + rows[t, pl.ds(c * L, L)]
    acc_v[i, pl.ds(col0 + c * L, L)] = s


def _side(tok_h, emb_h, idx_s, n_tok, hi_tok, acc_v, col0,
          toks, rowss, tsems, rsems):
  for b in range(NBUF):
    pltpu.async_copy(tok_h.at[pl.ds(idx_s[b] * 24, n_tok)], toks[b], tsems[b])

  n_grp = BW // NBUF

  def grp(g, _):
    i0 = g * NBUF
    for b in range(NBUF):
      i = i0 + b
      pltpu.make_async_copy(
          tok_h.at[pl.ds(idx_s[i] * 24, n_tok)], toks[b], tsems[b]).wait()
      _clamp_tok(toks[b], n_tok, hi_tok)
      pltpu.async_copy(emb_h.at[toks[b]], rowss[b], rsems[b])
    for b in range(NBUF):
      i = i0 + b
      pltpu.make_async_copy(emb_h.at[toks[b]], rowss[b], rsems[b]).wait()
      _accum_elem(rowss[b], n_tok, acc_v, i, col0)
      nxt = i + NBUF

      @pl.when(nxt < BW)
      def _():
        pltpu.async_copy(tok_h.at[pl.ds(idx_s[nxt] * 24, n_tok)],
                         toks[b], tsems[b])
    return 0

  lax.fori_loop(0, n_grp, grp, 0)

  tail = BW % NBUF
  for b in range(tail):
    i = (BW // NBUF) * NBUF + b
    pltpu.make_async_copy(
        tok_h.at[pl.ds(idx_s[i] * 24, n_tok)], toks[b], tsems[b]).wait()
    _clamp_tok(toks[b], n_tok, hi_tok)
    pltpu.async_copy(emb_h.at[toks[b]], rowss[b], rsems[b]).wait()
    _accum_elem(rowss[b], n_tok, acc_v, i, col0)


def _pool_relations(rtok_h, remb_h, s, pooled_sh, rtok_v, pooled_v,
                    toks, rowss, rsems):
  start = jnp.minimum(s * RPT, NRELS - RPT)
  pltpu.sync_copy(rtok_h.at[pl.ds(start, RPT)], rtok_v)

  def copy_tok(r, b):
    for off in (0, REL_MAX_LEN - L):
      v = rtok_v[r, pl.ds(off, L)]
      toks[b][pl.ds(off, L)] = jnp.minimum(jnp.maximum(v, 0), NRELS - 1)

  for b in range(NBUF):
    copy_tok(b, b)
    pltpu.async_copy(remb_h.at[toks[b]], rowss[b], rsems[b])

  def grp(g, _):
    r0 = g * NBUF
    for b in range(NBUF):
      r = r0 + b
      pltpu.make_async_copy(remb_h.at[toks[b]], rowss[b], rsems[b]).wait()
      _accum_elem(rowss[b], REL_MAX_LEN, pooled_v, r, 0)
      nxt = r + NBUF

      @pl.when(nxt < RPT)
      def _():
        copy_tok(nxt, b)
        pltpu.async_copy(remb_h.at[toks[b]], rowss[b], rsems[b])
    return 0

  lax.fori_loop(0, RPT // NBUF, grp, 0)

  tail = RPT % NBUF
  for b in range(tail):
    r = (RPT // NBUF) * NBUF + b
    pltpu.make_async_copy(remb_h.at[toks[b]], rowss[b], rsems[b]).wait()
    _accum_elem(rowss[b], REL_MAX_LEN, pooled_v, r, 0)

  pltpu.sync_copy(pooled_v, pooled_sh.at[pl.ds(start, RPT)])


def _body(subj_h, rel_h, etok_h, rtok_h, eemb_h, remb_h, out_h,
          ids_sh, pooled_sh, sidx_v, ridx_v, sidx_s, acc_v,
          rtok_v, pooled_v, rrows_v, toks, rowss, tsems, rsems):
  c = lax.axis_index("c")
  s = lax.axis_index("s")
  wid = s * NUM_CORES + c
  base = wid * BW

  pltpu.sync_copy(subj_h.at[pl.ds(base, BW)], sidx_v)
  pltpu.sync_copy(rel_h.at[pl.ds(base, BW)], ridx_v)
  _clamp_ids(sidx_v, BW, 100000 - 1)
  _clamp_ids(ridx_v, BW, NRELS - 1)
  pltpu.sync_copy(sidx_v, ids_sh.at[s])
  pltpu.sync_copy(ids_sh.at[s], sidx_s)

  _pool_relations(rtok_h, remb_h, s, pooled_sh, rtok_v, pooled_v,
                  toks, rowss, rsems)

  _side(etok_h, eemb_h, sidx_s, ENT_MAX_LEN, 100000 - 1, acc_v, 0,
        toks, rowss, tsems, rsems)

  plsc.subcore_barrier()
  pltpu.async_copy(pooled_sh.at[ridx_v], rrows_v, tsems[0]).wait()

  def cp(i, _):
    for ch in range(DIM // L):
      acc_v[i, pl.ds(DIM + ch * L, L)] = rrows_v[i, pl.ds(ch * L, L)]
    return 0

  lax.fori_loop(0, BW, cp, 0)

  pltpu.sync_copy(acc_v, out_h.at[pl.ds(base, BW)])


@jax.jit
def kernel(subj, rel, entity_token_ids, relation_token_ids,
           entity_emb, relation_emb):
  etok_flat = jnp.pad(entity_token_ids, ((0, 0), (0, 4))).reshape(-1)
  mesh = plsc.VectorSubcoreMesh(core_axis_name="c", subcore_axis_name="s")
  run = pl.kernel(
      _body,
      out_type=jax.ShapeDtypeStruct((BATCH, 2 * DIM), jnp.float32),
      mesh=mesh,
      scratch_types=[
          pltpu.VMEM_SHARED((NUM_SUBCORES, BW), jnp.int32),
          pltpu.VMEM_SHARED((NRELS, DIM), jnp.float32),
          pltpu.VMEM((BW,), jnp.int32),
          pltpu.VMEM((BW,), jnp.int32),
          pltpu.SMEM((BW,), jnp.int32),
          pltpu.VMEM((BW, 2 * DIM), jnp.float32),
          pltpu.VMEM((RPT, REL_MAX_LEN), jnp.int32),
          pltpu.VMEM((RPT, DIM), jnp.float32),
          pltpu.VMEM((BW, DIM), jnp.float32),
          [pltpu.VMEM((ENT_MAX_LEN,), jnp.int32) for _ in range(NBUF)],
          [pltpu.VMEM((ENT_MAX_LEN, DIM), jnp.float32) for _ in range(NBUF)],
          [pltpu.SemaphoreType.DMA for _ in range(NBUF)],
          [pltpu.SemaphoreType.DMA for _ in range(NBUF)],
      ],
  )
  return run(subj, rel, etok_flat, relation_token_ids,
             entity_emb, relation_emb)


# speedup vs baseline: 1.0183x; 1.0183x over previous
import jax
import jax.numpy as jnp
from jax import lax
from jax.experimental import pallas as pl
from jax.experimental.pallas import tpu as pltpu
from jax.experimental.pallas import tpu_sc as plsc

ENT_MAX_LEN = 20
REL_MAX_LEN = 20
DIM = 128
BATCH = 4096

NUM_CORES = 2
NUM_SUBCORES = 16
NW = NUM_CORES * NUM_SUBCORES
BW = BATCH // NW
NBUF = 8
L = 16


def _clamp_ids(ref_1d, n, hi):
  for c in range(n // L):
    v = ref_1d[pl.ds(c * L, L)]
    ref_1d[pl.ds(c * L, L)] = jnp.minimum(jnp.maximum(v, 0), hi)


def _clamp_tok(tok, n_tok, hi):
  for off in (0, n_tok - L):
    v = tok[pl.ds(off, L)]
    tok[pl.ds(off, L)] = jnp.minimum(jnp.maximum(v, 0), hi)


def _accum_elem(rows, n_tok, acc_v, i, col0):
  for c in range(DIM // L):
    s = rows[0, pl.ds(c * L, L)]
    for t in range(1, n_tok):
      s = s + rows[t, pl.ds(c * L, L)]
    acc_v[i, pl.ds(col0 + c * L, L)] = s


def _side(tok_h, emb_h, idx_s, n_tok, hi_tok, acc_v, col0,
          toks, rowss, tsems, rsems):
  for b in range(NBUF):
    pltpu.async_copy(tok_h.at[idx_s[b]], toks[b], tsems[b])

  n_grp = BW // NBUF

  def grp(g, _):
    i0 = g * NBUF
    for b in range(NBUF):
      i = i0 + b
      pltpu.make_async_copy(tok_h.at[idx_s[i]], toks[b], tsems[b]).wait()
      _clamp_tok(toks[b], n_tok, hi_tok)
      pltpu.async_copy(emb_h.at[toks[b]], rowss[b], rsems[b])
    for b in range(NBUF):
      i = i0 + b
      pltpu.make_async_copy(emb_h.at[toks[b]], rowss[b], rsems[b]).wait()
      _accum_elem(rowss[b], n_tok, acc_v, i, col0)
      nxt = i + NBUF

      @pl.when(nxt < BW)
      def _():
        pltpu.async_copy(tok_h.at[idx_s[nxt]], toks[b], tsems[b])
    return 0

  lax.fori_loop(0, n_grp, grp, 0)

  tail = BW % NBUF
  for b in range(tail):
    i = (BW // NBUF) * NBUF + b
    pltpu.make_async_copy(tok_h.at[idx_s[i]], toks[b], tsems[b]).wait()
    _clamp_tok(toks[b], n_tok, hi_tok)
    pltpu.async_copy(emb_h.at[toks[b]], rowss[b], rsems[b]).wait()
    _accum_elem(rowss[b], n_tok, acc_v, i, col0)


def _body(subj_h, rel_h, etok_h, rtok_h, eemb_h, remb_h, out_h,
          ids_sh, sidx_v, ridx_v, sidx_s, ridx_s, acc_v,
          toks, rowss, tsems, rsems):
  c = lax.axis_index("c")
  s = lax.axis_index("s")
  wid = s * NUM_CORES + c
  base = wid * BW

  pltpu.sync_copy(subj_h.at[pl.ds(base, BW)], sidx_v)
  pltpu.sync_copy(rel_h.at[pl.ds(base, BW)], ridx_v)
  _clamp_ids(sidx_v, BW, 100000 - 1)
  _clamp_ids(ridx_v, BW, 1000 - 1)
  pltpu.sync_copy(sidx_v, ids_sh.at[s, 0])
  pltpu.sync_copy(ridx_v, ids_sh.at[s, 1])
  pltpu.sync_copy(ids_sh.at[s, 0], sidx_s)
  pltpu.sync_copy(ids_sh.at[s, 1], ridx_s)

  _side(etok_h, eemb_h, sidx_s, ENT_MAX_LEN, 100000 - 1, acc_v, 0,
        toks, rowss, tsems, rsems)
  _side(rtok_h, remb_h, ridx_s, REL_MAX_LEN, 1000 - 1, acc_v, DIM,
        toks, rowss, tsems, rsems)

  pltpu.sync_copy(acc_v, out_h.at[pl.ds(base, BW)])


@jax.jit
def kernel(subj, rel, entity_token_ids, relation_token_ids,
           entity_emb, relation_emb):
  mesh = plsc.VectorSubcoreMesh(core_axis_name="c", subcore_axis_name="s")
  run = pl.kernel(
      _body,
      out_type=jax.ShapeDtypeStruct((BATCH, 2 * DIM), jnp.float32),
      mesh=mesh,
      scratch_types=[
          pltpu.VMEM_SHARED((NUM_SUBCORES, 2, BW), jnp.int32),
          pltpu.VMEM((BW,), jnp.int32),
          pltpu.VMEM((BW,), jnp.int32),
          pltpu.SMEM((BW,), jnp.int32),
          pltpu.SMEM((BW,), jnp.int32),
          pltpu.VMEM((BW, 2 * DIM), jnp.float32),
          [pltpu.VMEM((ENT_MAX_LEN,), jnp.int32) for _ in range(NBUF)],
          [pltpu.VMEM((ENT_MAX_LEN, DIM), jnp.float32) for _ in range(NBUF)],
          [pltpu.SemaphoreType.DMA for _ in range(NBUF)],
          [pltpu.SemaphoreType.DMA for _ in range(NBUF)],
      ],
  )
  return run(subj, rel, entity_token_ids, relation_token_ids,
             entity_emb, relation_emb)


# speedup vs baseline: 1.0415x; 1.0228x over previous
import jax
import jax.numpy as jnp
from jax import lax
from jax.experimental import pallas as pl
from jax.experimental.pallas import tpu as pltpu
from jax.experimental.pallas import tpu_sc as plsc

ENT_MAX_LEN = 20
REL_MAX_LEN = 20
DIM = 128
BATCH = 4096

NUM_CORES = 2
NUM_SUBCORES = 16
NW = NUM_CORES * NUM_SUBCORES
BW = BATCH // NW
NBUF = 6
L = 16


def _clamp_ids(ref_1d, n, hi):
  for c in range(n // L):
    v = ref_1d[pl.ds(c * L, L)]
    ref_1d[pl.ds(c * L, L)] = jnp.minimum(jnp.maximum(v, 0), hi)


def _clamp_tok(tok, n_tok, hi):
  for off in (0, n_tok - L):
    v = tok[pl.ds(off, L)]
    tok[pl.ds(off, L)] = jnp.minimum(jnp.maximum(v, 0), hi)


def _accum_elem(rows, n_tok, acc_v, i, col0):
  for c in range(DIM // L):
    s = rows[0, pl.ds(c * L, L)]
    for t in range(1, n_tok):
      s = s + rows[t, pl.ds(c * L, L)]
    acc_v[i, pl.ds(col0 + c * L, L)] = s


def _side(tok_h, emb_h, idx_s, n_tok, hi_tok, acc_v, col0,
          toks, rowss, tsems, rsems):
  for b in range(NBUF):
    pltpu.async_copy(tok_h.at[idx_s[b]], toks[b], tsems[b])

  n_grp = BW // NBUF

  def grp(g, _):
    i0 = g * NBUF
    for b in range(NBUF):
      i = i0 + b
      pltpu.make_async_copy(tok_h.at[idx_s[i]], toks[b], tsems[b]).wait()
      _clamp_tok(toks[b], n_tok, hi_tok)
      pltpu.async_copy(emb_h.at[toks[b]], rowss[b], rsems[b])
    for b in range(NBUF):
      i = i0 + b
      pltpu.make_async_copy(emb_h.at[toks[b]], rowss[b], rsems[b]).wait()
      _accum_elem(rowss[b], n_tok, acc_v, i, col0)
      nxt = i + NBUF

      @pl.when(nxt < BW)
      def _():
        pltpu.async_copy(tok_h.at[idx_s[nxt]], toks[b], tsems[b])
    return 0

  lax.fori_loop(0, n_grp, grp, 0)

  tail = BW % NBUF
  for b in range(tail):
    i = (BW // NBUF) * NBUF + b
    pltpu.make_async_copy(tok_h.at[idx_s[i]], toks[b], tsems[b]).wait()
    _clamp_tok(toks[b], n_tok, hi_tok)
    pltpu.async_copy(emb_h.at[toks[b]], rowss[b], rsems[b]).wait()
    _accum_elem(rowss[b], n_tok, acc_v, i, col0)


def _body(subj_h, rel_h, etok_h, rtok_h, eemb_h, remb_h, out_h,
          ids_sh, sidx_v, ridx_v, sidx_s, ridx_s, acc_v,
          toks, rowss, tsems, rsems):
  c = lax.axis_index("c")
  s = lax.axis_index("s")
  wid = s * NUM_CORES + c
  base = wid * BW

  pltpu.sync_copy(subj_h.at[pl.ds(base, BW)], sidx_v)
  pltpu.sync_copy(rel_h.at[pl.ds(base, BW)], ridx_v)
  _clamp_ids(sidx_v, BW, 100000 - 1)
  _clamp_ids(ridx_v, BW, 1000 - 1)
  pltpu.sync_copy(sidx_v, ids_sh.at[s, 0])
  pltpu.sync_copy(ridx_v, ids_sh.at[s, 1])
  pltpu.sync_copy(ids_sh.at[s, 0], sidx_s)
  pltpu.sync_copy(ids_sh.at[s, 1], ridx_s)

  _side(etok_h, eemb_h, sidx_s, ENT_MAX_LEN, 100000 - 1, acc_v, 0,
        toks, rowss, tsems, rsems)
  _side(rtok_h, remb_h, ridx_s, REL_MAX_LEN, 1000 - 1, acc_v, DIM,
        toks, rowss, tsems, rsems)

  pltpu.sync_copy(acc_v, out_h.at[pl.ds(base, BW)])


@jax.jit
def kernel(subj, rel, entity_token_ids, relation_token_ids,
           entity_emb, relation_emb):
  mesh = plsc.VectorSubcoreMesh(core_axis_name="c", subcore_axis_name="s")
  run = pl.kernel(
      _body,
      out_type=jax.ShapeDtypeStruct((BATCH, 2 * DIM), jnp.float32),
      mesh=mesh,
      scratch_types=[
          pltpu.VMEM_SHARED((NUM_SUBCORES, 2, BW), jnp.int32),
          pltpu.VMEM((BW,), jnp.int32),
          pltpu.VMEM((BW,), jnp.int32),
          pltpu.SMEM((BW,), jnp.int32),
          pltpu.SMEM((BW,), jnp.int32),
          pltpu.VMEM((BW, 2 * DIM), jnp.float32),
          [pltpu.VMEM((ENT_MAX_LEN,), jnp.int32) for _ in range(NBUF)],
          [pltpu.VMEM((ENT_MAX_LEN, DIM), jnp.float32) for _ in range(NBUF)],
          [pltpu.SemaphoreType.DMA for _ in range(NBUF)],
          [pltpu.SemaphoreType.DMA for _ in range(NBUF)],
      ],
  )
  return run(subj, rel, entity_token_ids, relation_token_ids,
             entity_emb, relation_emb)
